# Initial kernel scaffold; baseline (speedup 1.0000x reference)
import jax
import jax.numpy as jnp
from jax.experimental import pallas as pl


def kernel(addr_src, addr_dst, edge_feat, h_local, t, W1_src, b1_src, W2_src, b2_src, W1_dst, b1_dst, W2_dst, b2_dst):
    raise NotImplementedError("write your pallas kernel here")


# speedup vs baseline: 2.6977x; 2.6977x over previous
import functools

import jax
import jax.numpy as jnp
from jax import lax
from jax.experimental import pallas as pl
from jax.experimental.pallas import tpu as pltpu
from jax.experimental.pallas import tpu_sc as plsc

N = 50000
E = 800000
D_EDGE = 16
OUT = 32
HID = 64

NC = 2
NS = 16
NW = NC * NS
CHUNK = 128
K = 8
KS = 4
EROWS = 6400
EPAD = EROWS * CHUNK
CW = EROWS // NW
NACC = 51200
TSLAB = NACC // NS
RSTAGE = 512

_f32 = jnp.float32


def _gather_body(h_hbm, asrc_hbm, adst_hbm, gs_hbm, gd_hbm,
                 idx_s, idx_d, rows_s, rows_d, sem_s, sem_d):
    c = lax.axis_index("c")
    s = lax.axis_index("s")
    wid = s * NC + c
    row0 = wid * CW

    @pl.loop(0, CW // K)
    def _(i):
        rb = row0 + i * K
        pltpu.sync_copy(asrc_hbm.at[pl.ds(rb, K)], idx_s)
        pltpu.sync_copy(adst_hbm.at[pl.ds(rb, K)], idx_d)
        cps = [pltpu.async_copy(h_hbm.at[idx_s.at[j]], rows_s.at[j], sem_s)
               for j in range(K)]
        cpd = [pltpu.async_copy(h_hbm.at[idx_d.at[j]], rows_d.at[j], sem_d)
               for j in range(K)]
        for cp in cps:
            cp.wait()
        for cp in cpd:
            cp.wait()
        pltpu.sync_copy(rows_s, gs_hbm.at[pl.ds(rb, K)])
        pltpu.sync_copy(rows_d, gd_hbm.at[pl.ds(rb, K)])


def _scatter_body(zeros_hbm, asrc_hbm, adst_hbm, ds_hbm, dd_hbm, out_hbm,
                  acc, idx, rows, sem):
    c = lax.axis_index("c")
    s = lax.axis_index("s")

    nfull = TSLAB // RSTAGE
    rem = TSLAB - nfull * RSTAGE
    slab0 = s * TSLAB
    pltpu.sync_copy(zeros_hbm, rows)
    for q in range(nfull):
        pltpu.sync_copy(rows, acc.at[pl.ds(slab0 + q * RSTAGE, RSTAGE)])
    if rem:
        pltpu.sync_copy(rows.at[pl.ds(0, rem)],
                        acc.at[pl.ds(slab0 + nfull * RSTAGE, rem)])

    plsc.subcore_barrier()

    half = EROWS // NC
    tpw = half // NS
    row0 = c * half + s * tpw

    @pl.loop(0, tpw // KS)
    def _(i):
        rb = row0 + i * KS
        e0 = rb * CHUNK
        pltpu.sync_copy(asrc_hbm.at[pl.ds(rb, KS)], idx)
        pltpu.sync_copy(ds_hbm.at[pl.ds(e0, KS * CHUNK)], rows)
        for j in range(KS):
            pltpu.sync_copy(rows.at[pl.ds(j * CHUNK, CHUNK)],
                            acc.at[idx.at[j]], add=True)
        pltpu.sync_copy(adst_hbm.at[pl.ds(rb, KS)], idx)
        pltpu.sync_copy(dd_hbm.at[pl.ds(e0, KS * CHUNK)], rows)
        for j in range(KS):
            pltpu.sync_copy(rows.at[pl.ds(j * CHUNK, CHUNK)],
                            acc.at[idx.at[j]], add=True)

    plsc.subcore_barrier()

    for q in range(nfull):
        pltpu.sync_copy(acc.at[pl.ds(slab0 + q * RSTAGE, RSTAGE)], rows)
        pltpu.sync_copy(rows, out_hbm.at[c, pl.ds(slab0 + q * RSTAGE, RSTAGE)])
    if rem:
        pltpu.sync_copy(acc.at[pl.ds(slab0 + nfull * RSTAGE, rem)],
                        rows.at[pl.ds(0, rem)])
        pltpu.sync_copy(rows.at[pl.ds(0, rem)],
                        out_hbm.at[c, pl.ds(slab0 + nfull * RSTAGE, rem)])


def _mlp_body(gs_ref, gd_ref, ef_ref, w1_ref, b1_ref, w2_ref, b2_ref,
              ds_ref, dd_ref):
    x = jnp.concatenate([gs_ref[...], gd_ref[...], ef_ref[...]], axis=1)
    h = jnp.tanh(jnp.dot(x, w1_ref[...], preferred_element_type=_f32)
                 + b1_ref[...])
    d = jnp.tanh(jnp.dot(h, w2_ref[...], preferred_element_type=_f32)
                 + b2_ref[...])
    ds_ref[...] = d[:, :OUT]
    dd_ref[...] = d[:, OUT:]


def _final_body(p_ref, o_ref):
    o_ref[...] = jnp.tanh(p_ref[0] + p_ref[1])


_SC_MESH = plsc.VectorSubcoreMesh(core_axis_name="c", subcore_axis_name="s",
                                  num_cores=NC, num_subcores=NS)

_gather_call = pl.kernel(
    _gather_body,
    out_type=(jax.ShapeDtypeStruct((EROWS, CHUNK, OUT), _f32),
              jax.ShapeDtypeStruct((EROWS, CHUNK, OUT), _f32)),
    mesh=_SC_MESH,
    scratch_types=[
        pltpu.VMEM((K, CHUNK), jnp.int32),
        pltpu.VMEM((K, CHUNK), jnp.int32),
        pltpu.VMEM((K, CHUNK, OUT), _f32),
        pltpu.VMEM((K, CHUNK, OUT), _f32),
        pltpu.SemaphoreType.DMA,
        pltpu.SemaphoreType.DMA,
    ],
    compiler_params=pltpu.CompilerParams(use_tc_tiling_on_sc=False),
)

_scatter_call = pl.kernel(
    _scatter_body,
    out_type=jax.ShapeDtypeStruct((NC, NACC, OUT), _f32),
    mesh=_SC_MESH,
    scratch_types=[
        pltpu.VMEM_SHARED((NACC, OUT), _f32),
        pltpu.VMEM((KS, CHUNK), jnp.int32),
        pltpu.VMEM((RSTAGE, OUT), _f32),
        pltpu.SemaphoreType.DMA,
    ],
    compiler_params=pltpu.CompilerParams(use_tc_tiling_on_sc=False),
)

BE = 8192

_mlp_call = pl.pallas_call(
    _mlp_body,
    grid=(EPAD // BE,),
    in_specs=[
        pl.BlockSpec((BE, OUT), lambda i: (i, 0)),
        pl.BlockSpec((BE, OUT), lambda i: (i, 0)),
        pl.BlockSpec((BE, D_EDGE), lambda i: (i, 0)),
        pl.BlockSpec((2 * OUT + D_EDGE, 2 * HID), lambda i: (0, 0)),
        pl.BlockSpec((1, 2 * HID), lambda i: (0, 0)),
        pl.BlockSpec((2 * HID, 2 * OUT), lambda i: (0, 0)),
        pl.BlockSpec((1, 2 * OUT), lambda i: (0, 0)),
    ],
    out_specs=[
        pl.BlockSpec((BE, OUT), lambda i: (i, 0)),
        pl.BlockSpec((BE, OUT), lambda i: (i, 0)),
    ],
    out_shape=[
        jax.ShapeDtypeStruct((EPAD, OUT), _f32),
        jax.ShapeDtypeStruct((EPAD, OUT), _f32),
    ],
)

BN = 2000

_final_call = pl.pallas_call(
    _final_body,
    grid=(N // BN,),
    in_specs=[pl.BlockSpec((NC, BN, OUT), lambda i: (0, i, 0))],
    out_specs=pl.BlockSpec((BN, OUT), lambda i: (i, 0)),
    out_shape=jax.ShapeDtypeStruct((N, OUT), _f32),
)


def kernel(addr_src, addr_dst, edge_feat, h_local, t,
           W1_src, b1_src, W2_src, b2_src,
           W1_dst, b1_dst, W2_dst, b2_dst):
    asrc = addr_src.astype(jnp.int32)
    adst = addr_dst.astype(jnp.int32)
    pad = EPAD - E
    pad0 = jnp.zeros((pad,), jnp.int32)
    padN = jnp.full((pad,), N, jnp.int32)
    asrc_g = jnp.concatenate([asrc, pad0]).reshape(EROWS, CHUNK)
    adst_g = jnp.concatenate([adst, pad0]).reshape(EROWS, CHUNK)
    asrc_s = jnp.concatenate([asrc, padN]).reshape(EROWS, CHUNK)
    adst_s = jnp.concatenate([adst, padN]).reshape(EROWS, CHUNK)

    IN80 = 2 * OUT + D_EDGE
    w1cat = jnp.concatenate([W1_src[:IN80], W1_dst[:IN80]], axis=1)
    b1eff = jnp.concatenate([b1_src + t[0] * W1_src[IN80],
                             b1_dst + t[0] * W1_dst[IN80]])[None]
    w2bd = jnp.zeros((2 * HID, 2 * OUT), _f32)
    w2bd = w2bd.at[:HID, :OUT].set(W2_src).at[HID:, OUT:].set(W2_dst)
    b2cat = jnp.concatenate([b2_src, b2_dst])[None]

    ef = jnp.concatenate([edge_feat, jnp.zeros((pad, D_EDGE), _f32)])

    gs3, gd3 = _gather_call(h_local, asrc_g, adst_g)
    gs = gs3.reshape(EPAD, OUT)
    gd = gd3.reshape(EPAD, OUT)

    ds, dd = _mlp_call(gs, gd, ef, w1cat, b1eff, w2bd, b2cat)

    zeros = jnp.zeros((RSTAGE, OUT), _f32)
    partial = _scatter_call(zeros, asrc_s, adst_s, ds, dd)
    return _final_call(partial)


# speedup vs baseline: 2.8438x; 1.0542x over previous
import functools

import jax
import jax.numpy as jnp
from jax import lax
from jax.experimental import pallas as pl
from jax.experimental.pallas import tpu as pltpu
from jax.experimental.pallas import tpu_sc as plsc

N = 50000
E = 800000
D_EDGE = 16
OUT = 32
HID = 64

NC = 2
NS = 16
NW = NC * NS
CHUNK = 128
K = 10
KS = 2
EROWS = 6400
EPAD = EROWS * CHUNK
CW = EROWS // NW
NACC = 51200
TSLAB = NACC // NS
RSTAGE = 256
NSS = 100

_f32 = jnp.float32


NSET = 20


def _gather_body(h_hbm, asrc_hbm, adst_hbm, gs_hbm, gd_hbm,
                 idx, rows_a, rows_b, sem_a, sem_b):
    c = lax.axis_index("c")
    s = lax.axis_index("s")
    wid = s * NC + c
    row0 = wid * CW
    sb = K * CHUNK

    def field(addr2d, out2d):
        pltpu.sync_copy(addr2d.at[pl.ds(row0, CW)], idx)

        def fire(buf, sem, st):
            for j in range(K):
                pltpu.async_copy(h_hbm.at[idx.at[st * K + j]],
                                 buf.at[pl.ds(j * CHUNK, CHUNK)], sem)

        def drain(buf, sem):
            pltpu.make_async_copy(out2d.at[pl.ds(0, sb)], buf, sem).wait()

        def store(buf, st):
            pltpu.sync_copy(buf, out2d.at[pl.ds(row0 * CHUNK + st * sb, sb)])

        fire(rows_a, sem_a, 0)

        @pl.loop(0, NSET // 2 - 1)
        def _(i):
            st = 2 * i
            fire(rows_b, sem_b, st + 1)
            drain(rows_a, sem_a)
            store(rows_a, st)
            fire(rows_a, sem_a, st + 2)
            drain(rows_b, sem_b)
            store(rows_b, st + 1)

        fire(rows_b, sem_b, NSET - 1)
        drain(rows_a, sem_a)
        store(rows_a, NSET - 2)
        drain(rows_b, sem_b)
        store(rows_b, NSET - 1)

    field(asrc_hbm, gs_hbm)
    field(adst_hbm, gd_hbm)


def _scatter_body(zeros_hbm, asrc_hbm, adst_hbm, ds_hbm, dd_hbm, out_hbm,
                  acc, idx_a, idx_b, rows_a, rows_b, sem_a, sem_b):
    c = lax.axis_index("c")
    s = lax.axis_index("s")

    nfull = TSLAB // RSTAGE
    rem = TSLAB - nfull * RSTAGE
    slab0 = s * TSLAB
    pltpu.sync_copy(zeros_hbm, rows_a)
    for q in range(nfull):
        pltpu.sync_copy(rows_a, acc.at[pl.ds(slab0 + q * RSTAGE, RSTAGE)])
    if rem:
        pltpu.sync_copy(rows_a.at[pl.ds(0, rem)],
                        acc.at[pl.ds(slab0 + nfull * RSTAGE, rem)])

    plsc.subcore_barrier()

    half = EROWS // NC
    tpw = half // NS
    row0 = c * half + s * tpw

    def field(addr2d, delta2d):
        def load(ib, rbuf, st):
            r = row0 + st * KS
            pltpu.sync_copy(addr2d.at[pl.ds(r, KS)], ib)
            pltpu.sync_copy(delta2d.at[pl.ds(r * CHUNK, KS * CHUNK)], rbuf)

        def fire(ib, rbuf, sem):
            return [pltpu.async_copy(rbuf.at[pl.ds(j * CHUNK, CHUNK)],
                                     acc.at[ib.at[j]], sem, add=True)
                    for j in range(KS)]

        load(idx_a, rows_a, 0)

        @pl.loop(0, NSS // 2 - 1)
        def _(i):
            st = 2 * i
            cps = fire(idx_a, rows_a, sem_a)
            load(idx_b, rows_b, st + 1)
            for cp in cps:
                cp.wait()
            cps = fire(idx_b, rows_b, sem_b)
            load(idx_a, rows_a, st + 2)
            for cp in cps:
                cp.wait()

        cps = fire(idx_a, rows_a, sem_a)
        load(idx_b, rows_b, NSS - 1)
        for cp in cps:
            cp.wait()
        cps = fire(idx_b, rows_b, sem_b)
        for cp in cps:
            cp.wait()

    field(asrc_hbm, ds_hbm)
    field(adst_hbm, dd_hbm)

    plsc.subcore_barrier()

    for q in range(nfull):
        pltpu.sync_copy(acc.at[pl.ds(slab0 + q * RSTAGE, RSTAGE)], rows_a)
        pltpu.sync_copy(rows_a,
                        out_hbm.at[c, pl.ds(slab0 + q * RSTAGE, RSTAGE)])
    if rem:
        pltpu.sync_copy(acc.at[pl.ds(slab0 + nfull * RSTAGE, rem)],
                        rows_a.at[pl.ds(0, rem)])
        pltpu.sync_copy(rows_a.at[pl.ds(0, rem)],
                        out_hbm.at[c, pl.ds(slab0 + nfull * RSTAGE, rem)])


def _mlp_body(gs_ref, gd_ref, ef_ref, w1_ref, b1_ref, w2_ref, b2_ref,
              ds_ref, dd_ref):
    x = jnp.concatenate([gs_ref[...], gd_ref[...], ef_ref[...]], axis=1)
    h = jnp.tanh(jnp.dot(x, w1_ref[...], preferred_element_type=_f32)
                 + b1_ref[...])
    d = jnp.tanh(jnp.dot(h, w2_ref[...], preferred_element_type=_f32)
                 + b2_ref[...])
    ds_ref[...] = d[:, :OUT]
    dd_ref[...] = d[:, OUT:]


def _final_body(p_ref, o_ref):
    o_ref[...] = jnp.tanh(p_ref[0] + p_ref[1])


_SC_MESH = plsc.VectorSubcoreMesh(core_axis_name="c", subcore_axis_name="s",
                                  num_cores=NC, num_subcores=NS)

_gather_call = pl.kernel(
    _gather_body,
    out_type=(jax.ShapeDtypeStruct((EPAD, OUT), _f32),
              jax.ShapeDtypeStruct((EPAD, OUT), _f32)),
    mesh=_SC_MESH,
    scratch_types=[
        pltpu.VMEM((CW, CHUNK), jnp.int32),
        pltpu.VMEM((K * CHUNK, OUT), _f32),
        pltpu.VMEM((K * CHUNK, OUT), _f32),
        pltpu.SemaphoreType.DMA,
        pltpu.SemaphoreType.DMA,
    ],
    compiler_params=pltpu.CompilerParams(use_tc_tiling_on_sc=False),
)

_scatter_call = pl.kernel(
    _scatter_body,
    out_type=jax.ShapeDtypeStruct((NC, NACC, OUT), _f32),
    mesh=_SC_MESH,
    scratch_types=[
        pltpu.VMEM_SHARED((NACC, OUT), _f32),
        pltpu.VMEM((KS, CHUNK), jnp.int32),
        pltpu.VMEM((KS, CHUNK), jnp.int32),
        pltpu.VMEM((KS * CHUNK, OUT), _f32),
        pltpu.VMEM((KS * CHUNK, OUT), _f32),
        pltpu.SemaphoreType.DMA,
        pltpu.SemaphoreType.DMA,
    ],
    compiler_params=pltpu.CompilerParams(use_tc_tiling_on_sc=False),
)

BE = 8192

_mlp_call = pl.pallas_call(
    _mlp_body,
    grid=(EPAD // BE,),
    in_specs=[
        pl.BlockSpec((BE, OUT), lambda i: (i, 0)),
        pl.BlockSpec((BE, OUT), lambda i: (i, 0)),
        pl.BlockSpec((BE, D_EDGE),
                     lambda i: (jnp.minimum(i, (E + BE - 1) // BE - 1), 0)),
        pl.BlockSpec((2 * OUT + D_EDGE, 2 * HID), lambda i: (0, 0)),
        pl.BlockSpec((1, 2 * HID), lambda i: (0, 0)),
        pl.BlockSpec((2 * HID, 2 * OUT), lambda i: (0, 0)),
        pl.BlockSpec((1, 2 * OUT), lambda i: (0, 0)),
    ],
    out_specs=[
        pl.BlockSpec((BE, OUT), lambda i: (i, 0)),
        pl.BlockSpec((BE, OUT), lambda i: (i, 0)),
    ],
    out_shape=[
        jax.ShapeDtypeStruct((EPAD, OUT), _f32),
        jax.ShapeDtypeStruct((EPAD, OUT), _f32),
    ],
)

BN = 2000

_final_call = pl.pallas_call(
    _final_body,
    grid=(N // BN,),
    in_specs=[pl.BlockSpec((NC, BN, OUT), lambda i: (0, i, 0))],
    out_specs=pl.BlockSpec((BN, OUT), lambda i: (i, 0)),
    out_shape=jax.ShapeDtypeStruct((N, OUT), _f32),
)


def kernel(addr_src, addr_dst, edge_feat, h_local, t,
           W1_src, b1_src, W2_src, b2_src,
           W1_dst, b1_dst, W2_dst, b2_dst):
    asrc = addr_src.astype(jnp.int32)
    adst = addr_dst.astype(jnp.int32)
    pad = EPAD - E
    pad0 = jnp.zeros((pad,), jnp.int32)
    padN = jnp.full((pad,), N, jnp.int32)
    asrc_g = jnp.concatenate([asrc, pad0]).reshape(EROWS, CHUNK)
    adst_g = jnp.concatenate([adst, pad0]).reshape(EROWS, CHUNK)
    asrc_s = jnp.concatenate([asrc, padN]).reshape(EROWS, CHUNK)
    adst_s = jnp.concatenate([adst, padN]).reshape(EROWS, CHUNK)

    IN80 = 2 * OUT + D_EDGE
    w1cat = jnp.concatenate([W1_src[:IN80], W1_dst[:IN80]], axis=1)
    b1eff = jnp.concatenate([b1_src + t[0] * W1_src[IN80],
                             b1_dst + t[0] * W1_dst[IN80]])[None]
    w2bd = jnp.zeros((2 * HID, 2 * OUT), _f32)
    w2bd = w2bd.at[:HID, :OUT].set(W2_src).at[HID:, OUT:].set(W2_dst)
    b2cat = jnp.concatenate([b2_src, b2_dst])[None]

    gs, gd = _gather_call(h_local, asrc_g, adst_g)

    ds, dd = _mlp_call(gs, gd, edge_feat, w1cat, b1eff, w2bd, b2cat)

    zeros = jnp.zeros((RSTAGE, OUT), _f32)
    partial = _scatter_call(zeros, asrc_s, adst_s, ds, dd)
    return _final_call(partial)


# speedup vs baseline: 3.8395x; 1.3501x over previous
import functools

import jax
import jax.numpy as jnp
from jax import lax
from jax.experimental import pallas as pl
from jax.experimental.pallas import tpu as pltpu
from jax.experimental.pallas import tpu_sc as plsc

N = 50000
E = 800000
D_EDGE = 16
OUT = 32
HID = 64

NC = 2
NS = 16
NW = NC * NS
CHUNK = 128
K = 10
KS = 2
EROWS = 6400
EPAD = EROWS * CHUNK
CW = EROWS // NW
NACC = 51200
TSLAB = NACC // NS
RSTAGE = 256
NSS = 100

_f32 = jnp.float32


NSET = 20


def _gather_body(h_hbm, asrc_hbm, adst_hbm, gs_hbm, gd_hbm,
                 idx, rows_a, rows_b, sem_a, sem_b):
    c = lax.axis_index("c")
    s = lax.axis_index("s")
    wid = s * NC + c
    row0 = wid * CW
    sb = K * CHUNK

    def field(addr2d, out2d):
        pltpu.sync_copy(addr2d.at[pl.ds(row0, CW)], idx)

        def fire(buf, sem, st):
            for j in range(K):
                pltpu.async_copy(h_hbm.at[idx.at[st * K + j]],
                                 buf.at[pl.ds(j * CHUNK, CHUNK)], sem)

        def drain(buf, sem):
            pltpu.make_async_copy(out2d.at[pl.ds(0, sb)], buf, sem).wait()

        def store(buf, st):
            pltpu.sync_copy(buf, out2d.at[pl.ds(row0 * CHUNK + st * sb, sb)])

        fire(rows_a, sem_a, 0)

        @pl.loop(0, NSET // 2 - 1)
        def _(i):
            st = 2 * i
            fire(rows_b, sem_b, st + 1)
            drain(rows_a, sem_a)
            store(rows_a, st)
            fire(rows_a, sem_a, st + 2)
            drain(rows_b, sem_b)
            store(rows_b, st + 1)

        fire(rows_b, sem_b, NSET - 1)
        drain(rows_a, sem_a)
        store(rows_a, NSET - 2)
        drain(rows_b, sem_b)
        store(rows_b, NSET - 1)

    field(asrc_hbm, gs_hbm)
    field(adst_hbm, gd_hbm)


def _scatter_body(zeros_hbm, asrc_hbm, adst_hbm, ds_hbm, dd_hbm, out_hbm,
                  acc, idx_a, idx_b, rows_a, rows_b, sem_a, sem_b):
    c = lax.axis_index("c")
    s = lax.axis_index("s")

    nfull = TSLAB // RSTAGE
    rem = TSLAB - nfull * RSTAGE
    slab0 = s * TSLAB
    pltpu.sync_copy(zeros_hbm, rows_a)
    for q in range(nfull):
        pltpu.sync_copy(rows_a, acc.at[pl.ds(slab0 + q * RSTAGE, RSTAGE)])
    if rem:
        pltpu.sync_copy(rows_a.at[pl.ds(0, rem)],
                        acc.at[pl.ds(slab0 + nfull * RSTAGE, rem)])

    plsc.subcore_barrier()

    half = EROWS // NC
    tpw = half // NS
    row0 = c * half + s * tpw

    def field(addr2d, delta2d):
        def load(ib, rbuf, st):
            r = row0 + st * KS
            pltpu.sync_copy(addr2d.at[pl.ds(r, KS)], ib)
            pltpu.sync_copy(delta2d.at[pl.ds(r * CHUNK, KS * CHUNK)], rbuf)

        def fire(ib, rbuf, sem):
            return [pltpu.async_copy(rbuf.at[pl.ds(j * CHUNK, CHUNK)],
                                     acc.at[ib.at[j]], sem, add=True)
                    for j in range(KS)]

        load(idx_a, rows_a, 0)

        @pl.loop(0, NSS // 2 - 1)
        def _(i):
            st = 2 * i
            cps = fire(idx_a, rows_a, sem_a)
            load(idx_b, rows_b, st + 1)
            for cp in cps:
                cp.wait()
            cps = fire(idx_b, rows_b, sem_b)
            load(idx_a, rows_a, st + 2)
            for cp in cps:
                cp.wait()

        cps = fire(idx_a, rows_a, sem_a)
        load(idx_b, rows_b, NSS - 1)
        for cp in cps:
            cp.wait()
        cps = fire(idx_b, rows_b, sem_b)
        for cp in cps:
            cp.wait()

    field(asrc_hbm, ds_hbm)
    field(adst_hbm, dd_hbm)

    plsc.subcore_barrier()

    for q in range(nfull):
        pltpu.sync_copy(acc.at[pl.ds(slab0 + q * RSTAGE, RSTAGE)], rows_a)
        pltpu.sync_copy(rows_a,
                        out_hbm.at[c, pl.ds(slab0 + q * RSTAGE, RSTAGE)])
    if rem:
        pltpu.sync_copy(acc.at[pl.ds(slab0 + nfull * RSTAGE, rem)],
                        rows_a.at[pl.ds(0, rem)])
        pltpu.sync_copy(rows_a.at[pl.ds(0, rem)],
                        out_hbm.at[c, pl.ds(slab0 + nfull * RSTAGE, rem)])


def _mlp_body(gs_ref, gd_ref, ef_ref, w1_ref, b1_ref, w2_ref, b2_ref,
              ds_ref, dd_ref):
    q4 = BE // 4
    gs = jnp.concatenate([gs_ref[:, OUT * c:OUT * (c + 1)] for c in range(4)],
                         axis=0)
    gd = jnp.concatenate([gd_ref[:, OUT * c:OUT * (c + 1)] for c in range(4)],
                         axis=0)
    ef = jnp.concatenate(
        [ef_ref[:, D_EDGE * v:D_EDGE * (v + 1)] for v in range(8)], axis=0)
    x = jnp.concatenate([gs, gd, ef], axis=1)
    h = jnp.tanh(jnp.dot(x, w1_ref[...], preferred_element_type=_f32)
                 + b1_ref[...])
    d = jnp.tanh(jnp.dot(h, w2_ref[...], preferred_element_type=_f32)
                 + b2_ref[...])
    ds_ref[...] = jnp.concatenate(
        [d[q4 * c:q4 * (c + 1), :OUT] for c in range(4)], axis=1)
    dd_ref[...] = jnp.concatenate(
        [d[q4 * c:q4 * (c + 1), OUT:] for c in range(4)], axis=1)


def _final_body(p0_ref, p1_ref, o_ref):
    o_ref[...] = jnp.tanh(p0_ref[...] + p1_ref[...])


_SC_MESH = plsc.VectorSubcoreMesh(core_axis_name="c", subcore_axis_name="s",
                                  num_cores=NC, num_subcores=NS)

_gather_call = pl.kernel(
    _gather_body,
    out_type=(jax.ShapeDtypeStruct((EPAD, OUT), _f32),
              jax.ShapeDtypeStruct((EPAD, OUT), _f32)),
    mesh=_SC_MESH,
    scratch_types=[
        pltpu.VMEM((CW, CHUNK), jnp.int32),
        pltpu.VMEM((K * CHUNK, OUT), _f32),
        pltpu.VMEM((K * CHUNK, OUT), _f32),
        pltpu.SemaphoreType.DMA,
        pltpu.SemaphoreType.DMA,
    ],
    compiler_params=pltpu.CompilerParams(use_tc_tiling_on_sc=False),
)

_scatter_call = pl.kernel(
    _scatter_body,
    out_type=jax.ShapeDtypeStruct((NC, NACC, OUT), _f32),
    mesh=_SC_MESH,
    scratch_types=[
        pltpu.VMEM_SHARED((NACC, OUT), _f32),
        pltpu.VMEM((KS, CHUNK), jnp.int32),
        pltpu.VMEM((KS, CHUNK), jnp.int32),
        pltpu.VMEM((KS * CHUNK, OUT), _f32),
        pltpu.VMEM((KS * CHUNK, OUT), _f32),
        pltpu.SemaphoreType.DMA,
        pltpu.SemaphoreType.DMA,
    ],
    compiler_params=pltpu.CompilerParams(use_tc_tiling_on_sc=False),
)

BE = 8192

_mlp_call = pl.pallas_call(
    _mlp_body,
    grid=(EPAD // BE,),
    in_specs=[
        pl.BlockSpec((BE // 4, 128), lambda i: (i, 0)),
        pl.BlockSpec((BE // 4, 128), lambda i: (i, 0)),
        pl.BlockSpec((BE // 8, 128), lambda i: (i, 0)),
        pl.BlockSpec((2 * OUT + D_EDGE, 2 * HID), lambda i: (0, 0)),
        pl.BlockSpec((1, 2 * HID), lambda i: (0, 0)),
        pl.BlockSpec((2 * HID, 2 * OUT), lambda i: (0, 0)),
        pl.BlockSpec((1, 2 * OUT), lambda i: (0, 0)),
    ],
    out_specs=[
        pl.BlockSpec((BE // 4, 128), lambda i: (i, 0)),
        pl.BlockSpec((BE // 4, 128), lambda i: (i, 0)),
    ],
    out_shape=[
        jax.ShapeDtypeStruct((EPAD // 4, 128), _f32),
        jax.ShapeDtypeStruct((EPAD // 4, 128), _f32),
    ],
)

NAP = NACC * OUT // 128

_final_call = pl.pallas_call(
    _final_body,
    grid=(1,),
    in_specs=[
        pl.BlockSpec((NAP, 128), lambda i: (0, 0)),
        pl.BlockSpec((NAP, 128), lambda i: (1, 0)),
    ],
    out_specs=pl.BlockSpec((NAP, 128), lambda i: (0, 0)),
    out_shape=jax.ShapeDtypeStruct((NAP, 128), _f32),
)


def kernel(addr_src, addr_dst, edge_feat, h_local, t,
           W1_src, b1_src, W2_src, b2_src,
           W1_dst, b1_dst, W2_dst, b2_dst):
    asrc = addr_src.astype(jnp.int32)
    adst = addr_dst.astype(jnp.int32)
    pad = EPAD - E
    pad0 = jnp.zeros((pad,), jnp.int32)
    padN = jnp.full((pad,), N, jnp.int32)
    asrc_g = jnp.concatenate([asrc, pad0]).reshape(EROWS, CHUNK)
    adst_g = jnp.concatenate([adst, pad0]).reshape(EROWS, CHUNK)
    asrc_s = jnp.concatenate([asrc, padN]).reshape(EROWS, CHUNK)
    adst_s = jnp.concatenate([adst, padN]).reshape(EROWS, CHUNK)

    IN80 = 2 * OUT + D_EDGE
    w1cat = jnp.concatenate([W1_src[:IN80], W1_dst[:IN80]], axis=1)
    b1eff = jnp.concatenate([b1_src + t[0] * W1_src[IN80],
                             b1_dst + t[0] * W1_dst[IN80]])[None]
    w2bd = jnp.zeros((2 * HID, 2 * OUT), _f32)
    w2bd = w2bd.at[:HID, :OUT].set(W2_src).at[HID:, OUT:].set(W2_dst)
    b2cat = jnp.concatenate([b2_src, b2_dst])[None]

    nb = EPAD // BE
    ef_pad = jnp.concatenate([edge_feat, jnp.zeros((pad, D_EDGE), _f32)])
    ef4 = (ef_pad.reshape(nb, 2, BE // 8, 4, D_EDGE)
           .transpose(0, 2, 3, 1, 4).reshape(EPAD // 8, 128))

    gs, gd = _gather_call(h_local, asrc_g, adst_g)

    ds4, dd4 = _mlp_call(gs.reshape(EPAD // 4, 128), gd.reshape(EPAD // 4, 128),
                         ef4, w1cat, b1eff, w2bd, b2cat)

    zeros = jnp.zeros((RSTAGE, OUT), _f32)
    partial = _scatter_call(zeros, asrc_s, adst_s,
                            ds4.reshape(EPAD, OUT), dd4.reshape(EPAD, OUT))
    pp = partial.reshape(2 * NAP, 128)
    out4 = _final_call(pp, pp)
    return out4.reshape(NAP * 4, OUT)[:N]


# speedup vs baseline: 4.2389x; 1.1040x over previous
import functools

import jax
import jax.numpy as jnp
from jax import lax
from jax.experimental import pallas as pl
from jax.experimental.pallas import tpu as pltpu
from jax.experimental.pallas import tpu_sc as plsc

N = 50000
E = 800000
D_EDGE = 16
OUT = 32
HID = 64

NC = 2
NS = 16
NW = NC * NS
CHUNK = 128
K = 10
KS = 2
EROWS = 6400
EPAD = EROWS * CHUNK
CW = EROWS // NW
NACC = 51200
TSLAB = NACC // NS
RSTAGE = 256
NSS = 100

_f32 = jnp.float32


NSET = 20


def _gather_body(h_hbm, asrc_hbm, adst_hbm, gs_hbm, gd_hbm,
                 idx, rows_a, rows_b, sem_a, sem_b):
    c = lax.axis_index("c")
    s = lax.axis_index("s")
    wid = s * NC + c
    row0 = wid * CW
    sb = K * CHUNK

    def field(addr2d, out2d):
        pltpu.sync_copy(addr2d.at[pl.ds(row0, CW)], idx)

        def fire(buf, sem, st):
            for j in range(K):
                pltpu.async_copy(h_hbm.at[idx.at[st * K + j]],
                                 buf.at[pl.ds(j * CHUNK, CHUNK)], sem)

        def drain(buf, sem):
            pltpu.make_async_copy(out2d.at[pl.ds(0, sb)], buf, sem).wait()

        def store(buf, st):
            pltpu.sync_copy(buf, out2d.at[pl.ds(row0 * CHUNK + st * sb, sb)])

        fire(rows_a, sem_a, 0)

        @pl.loop(0, NSET // 2 - 1)
        def _(i):
            st = 2 * i
            fire(rows_b, sem_b, st + 1)
            drain(rows_a, sem_a)
            store(rows_a, st)
            fire(rows_a, sem_a, st + 2)
            drain(rows_b, sem_b)
            store(rows_b, st + 1)

        fire(rows_b, sem_b, NSET - 1)
        drain(rows_a, sem_a)
        store(rows_a, NSET - 2)
        drain(rows_b, sem_b)
        store(rows_b, NSET - 1)

    field(asrc_hbm, gs_hbm)
    field(adst_hbm, gd_hbm)


def _scatter_body(zeros_hbm, asrc_hbm, adst_hbm, ds_hbm, dd_hbm, out_hbm,
                  acc, idx_a, idx_b, rows_a, rows_b, sem_a, sem_b):
    c = lax.axis_index("c")
    s = lax.axis_index("s")

    nfull = TSLAB // RSTAGE
    rem = TSLAB - nfull * RSTAGE
    slab0 = s * TSLAB
    pltpu.sync_copy(zeros_hbm, rows_a)
    for q in range(nfull):
        pltpu.sync_copy(rows_a, acc.at[pl.ds(slab0 + q * RSTAGE, RSTAGE)])
    if rem:
        pltpu.sync_copy(rows_a.at[pl.ds(0, rem)],
                        acc.at[pl.ds(slab0 + nfull * RSTAGE, rem)])

    plsc.subcore_barrier()

    half = EROWS // NC
    tpw = half // NS
    row0 = c * half + s * tpw

    def field(addr2d, delta2d):
        def load(ib, rbuf, st):
            r = row0 + st * KS
            pltpu.sync_copy(addr2d.at[pl.ds(r, KS)], ib)
            pltpu.sync_copy(delta2d.at[pl.ds(r * CHUNK, KS * CHUNK)], rbuf)

        def fire(ib, rbuf, sem):
            return [pltpu.async_copy(rbuf.at[pl.ds(j * CHUNK, CHUNK)],
                                     acc.at[ib.at[j]], sem, add=True)
                    for j in range(KS)]

        load(idx_a, rows_a, 0)

        @pl.loop(0, NSS // 2 - 1)
        def _(i):
            st = 2 * i
            cps = fire(idx_a, rows_a, sem_a)
            load(idx_b, rows_b, st + 1)
            for cp in cps:
                cp.wait()
            cps = fire(idx_b, rows_b, sem_b)
            load(idx_a, rows_a, st + 2)
            for cp in cps:
                cp.wait()

        cps = fire(idx_a, rows_a, sem_a)
        load(idx_b, rows_b, NSS - 1)
        for cp in cps:
            cp.wait()
        cps = fire(idx_b, rows_b, sem_b)
        for cp in cps:
            cp.wait()

    field(asrc_hbm, ds_hbm)
    field(adst_hbm, dd_hbm)

    plsc.subcore_barrier()

    for q in range(nfull):
        pltpu.sync_copy(acc.at[pl.ds(slab0 + q * RSTAGE, RSTAGE)], rows_a)
        pltpu.sync_copy(rows_a,
                        out_hbm.at[c, pl.ds(slab0 + q * RSTAGE, RSTAGE)])
    if rem:
        pltpu.sync_copy(acc.at[pl.ds(slab0 + nfull * RSTAGE, rem)],
                        rows_a.at[pl.ds(0, rem)])
        pltpu.sync_copy(rows_a.at[pl.ds(0, rem)],
                        out_hbm.at[c, pl.ds(slab0 + nfull * RSTAGE, rem)])


def _mlp_body(gs_ref, gd_ref, eft_ref, w1ab_ref, w1c_ref, b1_ref, w2_ref,
              b2_ref, ds_ref, dd_ref):
    q4 = BE // 4
    gs = jnp.concatenate([gs_ref[:, OUT * c:OUT * (c + 1)] for c in range(4)],
                         axis=0)
    gd = jnp.concatenate([gd_ref[:, OUT * c:OUT * (c + 1)] for c in range(4)],
                         axis=0)
    xg = jnp.concatenate([gs, gd], axis=1)
    hp = (jnp.dot(xg, w1ab_ref[...], preferred_element_type=_f32)
          + jax.lax.dot_general(eft_ref[...], w1c_ref[...],
                                (((0,), (0,)), ((), ())),
                                preferred_element_type=_f32))
    h = jnp.tanh(hp + b1_ref[...])
    d = jnp.tanh(jnp.dot(h, w2_ref[...], preferred_element_type=_f32)
                 + b2_ref[...])
    ds_ref[...] = jnp.concatenate(
        [d[q4 * c:q4 * (c + 1), :OUT] for c in range(4)], axis=1)
    dd_ref[...] = jnp.concatenate(
        [d[q4 * c:q4 * (c + 1), OUT:] for c in range(4)], axis=1)


def _final_body(p0_ref, p1_ref, o_ref):
    o_ref[...] = jnp.tanh(p0_ref[...] + p1_ref[...])


_SC_MESH = plsc.VectorSubcoreMesh(core_axis_name="c", subcore_axis_name="s",
                                  num_cores=NC, num_subcores=NS)

_gather_call = pl.kernel(
    _gather_body,
    out_type=(jax.ShapeDtypeStruct((EPAD, OUT), _f32),
              jax.ShapeDtypeStruct((EPAD, OUT), _f32)),
    mesh=_SC_MESH,
    scratch_types=[
        pltpu.VMEM((CW, CHUNK), jnp.int32),
        pltpu.VMEM((K * CHUNK, OUT), _f32),
        pltpu.VMEM((K * CHUNK, OUT), _f32),
        pltpu.SemaphoreType.DMA,
        pltpu.SemaphoreType.DMA,
    ],
    compiler_params=pltpu.CompilerParams(use_tc_tiling_on_sc=False),
)

_scatter_call = pl.kernel(
    _scatter_body,
    out_type=jax.ShapeDtypeStruct((NC, NACC, OUT), _f32),
    mesh=_SC_MESH,
    scratch_types=[
        pltpu.VMEM_SHARED((NACC, OUT), _f32),
        pltpu.VMEM((KS, CHUNK), jnp.int32),
        pltpu.VMEM((KS, CHUNK), jnp.int32),
        pltpu.VMEM((KS * CHUNK, OUT), _f32),
        pltpu.VMEM((KS * CHUNK, OUT), _f32),
        pltpu.SemaphoreType.DMA,
        pltpu.SemaphoreType.DMA,
    ],
    compiler_params=pltpu.CompilerParams(use_tc_tiling_on_sc=False),
)

BE = 8192

_mlp_call = pl.pallas_call(
    _mlp_body,
    grid=(EPAD // BE,),
    in_specs=[
        pl.BlockSpec((BE // 4, 128), lambda i: (i, 0)),
        pl.BlockSpec((BE // 4, 128), lambda i: (i, 0)),
        pl.BlockSpec((D_EDGE, BE), lambda i: (0, i)),
        pl.BlockSpec((2 * OUT, 2 * HID), lambda i: (0, 0)),
        pl.BlockSpec((D_EDGE, 2 * HID), lambda i: (0, 0)),
        pl.BlockSpec((1, 2 * HID), lambda i: (0, 0)),
        pl.BlockSpec((2 * HID, 2 * OUT), lambda i: (0, 0)),
        pl.BlockSpec((1, 2 * OUT), lambda i: (0, 0)),
    ],
    out_specs=[
        pl.BlockSpec((BE // 4, 128), lambda i: (i, 0)),
        pl.BlockSpec((BE // 4, 128), lambda i: (i, 0)),
    ],
    out_shape=[
        jax.ShapeDtypeStruct((EPAD // 4, 128), _f32),
        jax.ShapeDtypeStruct((EPAD // 4, 128), _f32),
    ],
)

NAP = NACC * OUT // 128

_final_call = pl.pallas_call(
    _final_body,
    grid=(1,),
    in_specs=[
        pl.BlockSpec((NAP, 128), lambda i: (0, 0)),
        pl.BlockSpec((NAP, 128), lambda i: (1, 0)),
    ],
    out_specs=pl.BlockSpec((NAP, 128), lambda i: (0, 0)),
    out_shape=jax.ShapeDtypeStruct((NAP, 128), _f32),
)


def kernel(addr_src, addr_dst, edge_feat, h_local, t,
           W1_src, b1_src, W2_src, b2_src,
           W1_dst, b1_dst, W2_dst, b2_dst):
    asrc = addr_src.astype(jnp.int32)
    adst = addr_dst.astype(jnp.int32)
    pad = EPAD - E
    pad0 = jnp.zeros((pad,), jnp.int32)
    padN = jnp.full((pad,), N, jnp.int32)
    asrc_g = jnp.concatenate([asrc, pad0]).reshape(EROWS, CHUNK)
    adst_g = jnp.concatenate([adst, pad0]).reshape(EROWS, CHUNK)
    asrc_s = jnp.concatenate([asrc, padN]).reshape(EROWS, CHUNK)
    adst_s = jnp.concatenate([adst, padN]).reshape(EROWS, CHUNK)

    IN80 = 2 * OUT + D_EDGE
    w1cat = jnp.concatenate([W1_src[:IN80], W1_dst[:IN80]], axis=1)
    b1eff = jnp.concatenate([b1_src + t[0] * W1_src[IN80],
                             b1_dst + t[0] * W1_dst[IN80]])[None]
    w2bd = jnp.zeros((2 * HID, 2 * OUT), _f32)
    w2bd = w2bd.at[:HID, :OUT].set(W2_src).at[HID:, OUT:].set(W2_dst)
    b2cat = jnp.concatenate([b2_src, b2_dst])[None]

    nb = EPAD // BE
    eft = jnp.concatenate([edge_feat.T, jnp.zeros((D_EDGE, pad), _f32)],
                          axis=1)
    eftp = (eft.reshape(D_EDGE, nb, BE // 4, 4)
            .transpose(0, 1, 3, 2).reshape(D_EDGE, EPAD))

    gs, gd = _gather_call(h_local, asrc_g, adst_g)

    ds4, dd4 = _mlp_call(gs.reshape(EPAD // 4, 128), gd.reshape(EPAD // 4, 128),
                         eftp, w1cat[:2 * OUT], w1cat[2 * OUT:],
                         b1eff, w2bd, b2cat)

    zeros = jnp.zeros((RSTAGE, OUT), _f32)
    partial = _scatter_call(zeros, asrc_s, adst_s,
                            ds4.reshape(EPAD, OUT), dd4.reshape(EPAD, OUT))
    pp = partial.reshape(2 * NAP, 128)
    out4 = _final_call(pp, pp)
    return out4.reshape(NAP * 4, OUT)[:N]


# speedup vs baseline: 4.4273x; 1.0445x over previous
import functools

import jax
import jax.numpy as jnp
from jax import lax
from jax.experimental import pallas as pl
from jax.experimental.pallas import tpu as pltpu
from jax.experimental.pallas import tpu_sc as plsc

N = 50000
E = 800000
D_EDGE = 16
OUT = 32
HID = 64

NC = 2
NS = 16
NW = NC * NS
CHUNK = 128
K = 10
KS = 2
EROWS = 6400
EPAD = EROWS * CHUNK
CW = EROWS // NW
NACC = 51200
TSLAB = NACC // NS
RSTAGE = 256
NSS = 100

_f32 = jnp.float32


EHALF = EROWS // 2
CWH = EHALF // NW
NSET = CWH // K


def _gather_body(h_hbm, asrc_hbm, adst_hbm, gs_hbm, gd_hbm,
                 idx, rows_a, rows_b, sem_a, sem_b):
    c = lax.axis_index("c")
    s = lax.axis_index("s")
    wid = s * NC + c
    row0 = wid * CWH
    sb = K * CHUNK

    def field(addr2d, out2d):
        pltpu.sync_copy(addr2d.at[pl.ds(row0, CWH)], idx)

        def fire(buf, sem, st):
            for j in range(K):
                pltpu.async_copy(h_hbm.at[idx.at[st * K + j]],
                                 buf.at[pl.ds(j * CHUNK, CHUNK)], sem)

        def drain(buf, sem):
            pltpu.make_async_copy(out2d.at[pl.ds(0, sb)], buf, sem).wait()

        def store(buf, st):
            pltpu.sync_copy(buf, out2d.at[pl.ds(row0 * CHUNK + st * sb, sb)])

        fire(rows_a, sem_a, 0)

        @pl.loop(0, NSET // 2 - 1)
        def _(i):
            st = 2 * i
            fire(rows_b, sem_b, st + 1)
            drain(rows_a, sem_a)
            store(rows_a, st)
            fire(rows_a, sem_a, st + 2)
            drain(rows_b, sem_b)
            store(rows_b, st + 1)

        fire(rows_b, sem_b, NSET - 1)
        drain(rows_a, sem_a)
        store(rows_a, NSET - 2)
        drain(rows_b, sem_b)
        store(rows_b, NSET - 1)

    field(asrc_hbm, gs_hbm)
    field(adst_hbm, gd_hbm)


def _scatter_body(zeros_hbm, as1_hbm, ad1_hbm, as2_hbm, ad2_hbm,
                  ds1_hbm, dd1_hbm, ds2_hbm, dd2_hbm, out_hbm,
                  acc, idx_a, idx_b, rows_a, rows_b, sem_a, sem_b):
    c = lax.axis_index("c")
    s = lax.axis_index("s")

    nfull = TSLAB // RSTAGE
    rem = TSLAB - nfull * RSTAGE
    slab0 = s * TSLAB
    pltpu.sync_copy(zeros_hbm, rows_a)
    for q in range(nfull):
        pltpu.sync_copy(rows_a, acc.at[pl.ds(slab0 + q * RSTAGE, RSTAGE)])
    if rem:
        pltpu.sync_copy(rows_a.at[pl.ds(0, rem)],
                        acc.at[pl.ds(slab0 + nfull * RSTAGE, rem)])

    plsc.subcore_barrier()

    tpw = EHALF // NS
    row0 = s * tpw

    def field(addr2d, delta2d):
        def load(ib, rbuf, st):
            r = row0 + st * KS
            pltpu.sync_copy(addr2d.at[pl.ds(r, KS)], ib)
            pltpu.sync_copy(delta2d.at[pl.ds(r * CHUNK, KS * CHUNK)], rbuf)

        def fire(ib, rbuf, sem):
            return [pltpu.async_copy(rbuf.at[pl.ds(j * CHUNK, CHUNK)],
                                     acc.at[ib.at[j]], sem, add=True)
                    for j in range(KS)]

        load(idx_a, rows_a, 0)

        @pl.loop(0, NSS // 2 - 1)
        def _(i):
            st = 2 * i
            cps = fire(idx_a, rows_a, sem_a)
            load(idx_b, rows_b, st + 1)
            for cp in cps:
                cp.wait()
            cps = fire(idx_b, rows_b, sem_b)
            load(idx_a, rows_a, st + 2)
            for cp in cps:
                cp.wait()

        cps = fire(idx_a, rows_a, sem_a)
        load(idx_b, rows_b, NSS - 1)
        for cp in cps:
            cp.wait()
        cps = fire(idx_b, rows_b, sem_b)
        for cp in cps:
            cp.wait()

    @pl.when(c == 0)
    def _():
        field(as1_hbm, ds1_hbm)
        field(ad1_hbm, dd1_hbm)

    @pl.when(c == 1)
    def _():
        field(as2_hbm, ds2_hbm)
        field(ad2_hbm, dd2_hbm)

    plsc.subcore_barrier()

    for q in range(nfull):
        pltpu.sync_copy(acc.at[pl.ds(slab0 + q * RSTAGE, RSTAGE)], rows_a)
        pltpu.sync_copy(rows_a,
                        out_hbm.at[c, pl.ds(slab0 + q * RSTAGE, RSTAGE)])
    if rem:
        pltpu.sync_copy(acc.at[pl.ds(slab0 + nfull * RSTAGE, rem)],
                        rows_a.at[pl.ds(0, rem)])
        pltpu.sync_copy(rows_a.at[pl.ds(0, rem)],
                        out_hbm.at[c, pl.ds(slab0 + nfull * RSTAGE, rem)])


def _mlp_body(gs_ref, gd_ref, eft_ref, w1ab_ref, w1c_ref, b1_ref, w2_ref,
              b2_ref, ds_ref, dd_ref):
    q4 = BE // 4
    gs = jnp.concatenate([gs_ref[:, OUT * c:OUT * (c + 1)] for c in range(4)],
                         axis=0)
    gd = jnp.concatenate([gd_ref[:, OUT * c:OUT * (c + 1)] for c in range(4)],
                         axis=0)
    xg = jnp.concatenate([gs, gd], axis=1)
    hp = (jnp.dot(xg, w1ab_ref[...], preferred_element_type=_f32)
          + jax.lax.dot_general(eft_ref[...], w1c_ref[...],
                                (((0,), (0,)), ((), ())),
                                preferred_element_type=_f32))
    h = jnp.tanh(hp + b1_ref[...])
    d = jnp.tanh(jnp.dot(h, w2_ref[...], preferred_element_type=_f32)
                 + b2_ref[...])
    ds_ref[...] = jnp.concatenate(
        [d[q4 * c:q4 * (c + 1), :OUT] for c in range(4)], axis=1)
    dd_ref[...] = jnp.concatenate(
        [d[q4 * c:q4 * (c + 1), OUT:] for c in range(4)], axis=1)


def _final_body(p0_ref, p1_ref, o_ref):
    o_ref[...] = jnp.tanh(p0_ref[...] + p1_ref[...])


_SC_MESH = plsc.VectorSubcoreMesh(core_axis_name="c", subcore_axis_name="s",
                                  num_cores=NC, num_subcores=NS)

EH = EHALF * CHUNK

_gather_call = pl.kernel(
    _gather_body,
    out_type=(jax.ShapeDtypeStruct((EH, OUT), _f32),
              jax.ShapeDtypeStruct((EH, OUT), _f32)),
    mesh=_SC_MESH,
    scratch_types=[
        pltpu.VMEM((CWH, CHUNK), jnp.int32),
        pltpu.VMEM((K * CHUNK, OUT), _f32),
        pltpu.VMEM((K * CHUNK, OUT), _f32),
        pltpu.SemaphoreType.DMA,
        pltpu.SemaphoreType.DMA,
    ],
    compiler_params=pltpu.CompilerParams(use_tc_tiling_on_sc=False),
)

_scatter_call = pl.kernel(
    _scatter_body,
    out_type=jax.ShapeDtypeStruct((NC, NACC, OUT), _f32),
    mesh=_SC_MESH,
    scratch_types=[
        pltpu.VMEM_SHARED((NACC, OUT), _f32),
        pltpu.VMEM((KS, CHUNK), jnp.int32),
        pltpu.VMEM((KS, CHUNK), jnp.int32),
        pltpu.VMEM((KS * CHUNK, OUT), _f32),
        pltpu.VMEM((KS * CHUNK, OUT), _f32),
        pltpu.SemaphoreType.DMA,
        pltpu.SemaphoreType.DMA,
    ],
    compiler_params=pltpu.CompilerParams(use_tc_tiling_on_sc=False),
)

BE = 8192

_mlp_call = pl.pallas_call(
    _mlp_body,
    grid=(EH // BE,),
    in_specs=[
        pl.BlockSpec((BE // 4, 128), lambda i: (i, 0)),
        pl.BlockSpec((BE // 4, 128), lambda i: (i, 0)),
        pl.BlockSpec((D_EDGE, BE), lambda i: (0, i)),
        pl.BlockSpec((2 * OUT, 2 * HID), lambda i: (0, 0)),
        pl.BlockSpec((D_EDGE, 2 * HID), lambda i: (0, 0)),
        pl.BlockSpec((1, 2 * HID), lambda i: (0, 0)),
        pl.BlockSpec((2 * HID, 2 * OUT), lambda i: (0, 0)),
        pl.BlockSpec((1, 2 * OUT), lambda i: (0, 0)),
    ],
    out_specs=[
        pl.BlockSpec((BE // 4, 128), lambda i: (i, 0)),
        pl.BlockSpec((BE // 4, 128), lambda i: (i, 0)),
    ],
    out_shape=[
        jax.ShapeDtypeStruct((EH // 4, 128), _f32),
        jax.ShapeDtypeStruct((EH // 4, 128), _f32),
    ],
)

NAP = NACC * OUT // 128

_final_call = pl.pallas_call(
    _final_body,
    grid=(1,),
    in_specs=[
        pl.BlockSpec((NAP, 128), lambda i: (0, 0)),
        pl.BlockSpec((NAP, 128), lambda i: (1, 0)),
    ],
    out_specs=pl.BlockSpec((NAP, 128), lambda i: (0, 0)),
    out_shape=jax.ShapeDtypeStruct((NAP, 128), _f32),
)


def kernel(addr_src, addr_dst, edge_feat, h_local, t,
           W1_src, b1_src, W2_src, b2_src,
           W1_dst, b1_dst, W2_dst, b2_dst):
    asrc = addr_src.astype(jnp.int32)
    adst = addr_dst.astype(jnp.int32)
    pad = EPAD - E
    pad0 = jnp.zeros((pad,), jnp.int32)
    padN = jnp.full((pad,), N, jnp.int32)
    asrc_g = jnp.concatenate([asrc, pad0]).reshape(EROWS, CHUNK)
    adst_g = jnp.concatenate([adst, pad0]).reshape(EROWS, CHUNK)
    asrc_s = jnp.concatenate([asrc, padN]).reshape(EROWS, CHUNK)
    adst_s = jnp.concatenate([adst, padN]).reshape(EROWS, CHUNK)

    IN80 = 2 * OUT + D_EDGE
    w1cat = jnp.concatenate([W1_src[:IN80], W1_dst[:IN80]], axis=1)
    b1eff = jnp.concatenate([b1_src + t[0] * W1_src[IN80],
                             b1_dst + t[0] * W1_dst[IN80]])[None]
    w2bd = jnp.zeros((2 * HID, 2 * OUT), _f32)
    w2bd = w2bd.at[:HID, :OUT].set(W2_src).at[HID:, OUT:].set(W2_dst)
    b2cat = jnp.concatenate([b2_src, b2_dst])[None]

    nb = EPAD // BE
    eft = jnp.concatenate([edge_feat.T, jnp.zeros((D_EDGE, pad), _f32)],
                          axis=1)
    eftp = (eft.reshape(D_EDGE, nb, BE // 4, 4)
            .transpose(0, 1, 3, 2).reshape(D_EDGE, EPAD))

    w1ab, w1c = w1cat[:2 * OUT], w1cat[2 * OUT:]
    ef1, ef2 = eftp[:, :EH], eftp[:, EH:]
    outs = []
    for ck in range(2):
        sl = slice(ck * EHALF, (ck + 1) * EHALF)
        g1, g2 = _gather_call(h_local, asrc_g[sl], adst_g[sl])
        outs.append(_mlp_call(g1.reshape(EH // 4, 128),
                              g2.reshape(EH // 4, 128),
                              (ef1, ef2)[ck], w1ab, w1c, b1eff, w2bd, b2cat))
    (ds1, dd1), (ds2, dd2) = outs

    zeros = jnp.zeros((RSTAGE, OUT), _f32)
    partial = _scatter_call(zeros,
                            asrc_s[:EHALF], adst_s[:EHALF],
                            asrc_s[EHALF:], adst_s[EHALF:],
                            ds1.reshape(EH, OUT), dd1.reshape(EH, OUT),
                            ds2.reshape(EH, OUT), dd2.reshape(EH, OUT))
    pp = partial.reshape(2 * NAP, 128)
    out4 = _final_call(pp, pp)
    return out4.reshape(NAP * 4, OUT)[:N]
